# baseline (speedup 1.0000x reference)
import functools
import jax
import jax.numpy as jnp
from jax.experimental import pallas as pl


H = 8


def _mm_body(a_ref, w_ref, b_ref, o_ref, *, act):
    y = jnp.dot(a_ref[...], w_ref[...], preferred_element_type=jnp.float32)
    y = y + b_ref[...]
    if act == "relu":
        y = jnp.maximum(y, 0.0)
    elif act == "sigmoid":
        y = jax.nn.sigmoid(y)
    o_ref[...] = y


def _mm(a, w, b, act="none", block=2000):
    n, k = a.shape
    m = w.shape[1]
    assert n % block == 0, (n, block)
    return pl.pallas_call(
        functools.partial(_mm_body, act=act),
        grid=(n // block,),
        in_specs=[
            pl.BlockSpec((block, k), lambda i: (i, 0)),
            pl.BlockSpec((k, m), lambda i: (0, 0)),
            pl.BlockSpec((m,), lambda i: (0,)),
        ],
        out_specs=pl.BlockSpec((block, m), lambda i: (i, 0)),
        out_shape=jax.ShapeDtypeStruct((n, m), jnp.float32),
    )(a, w, b)


def _gat_conv_edges(xl_flat, a_src, a_dst, src, dst, heads, out_ch, n):
    alpha = a_src[src] + a_dst[dst]
    alpha = jax.nn.leaky_relu(alpha, 0.2)
    amax = jax.ops.segment_max(alpha, dst, num_segments=n)
    amax = jnp.where(jnp.isfinite(amax), amax, 0.0)
    ex = jnp.exp(alpha - amax[dst])
    denom = jax.ops.segment_sum(ex, dst, num_segments=n)
    coef = ex / (denom[dst] + 1e-16)
    msg = xl_flat.reshape(n, heads, out_ch)[src] * coef[:, :, None]
    out = jax.ops.segment_sum(msg, dst, num_segments=n)
    return out.reshape(n, heads * out_ch)


def _head_proj(att, out_ch):
    eye = jnp.eye(H, dtype=jnp.float32)
    return (att[:, :, None] * eye[:, None, :]).reshape(H * out_ch, H)


def kernel(x, edge_index, Wc1, as1, ad1, bc1, Wc2, as2, ad2, bc2, Wc3, as3, ad3, bc3, Wn1, bn1, Wn2, bn2, Wa1, ba1, Wa2, ba2, Wa3, ba3, Wa4, ba4, Wf1, bf1, Wf2, bf2, Wf3, bf3):
    n = x.shape[0]
    loops = jnp.arange(n, dtype=edge_index.dtype)
    src = jnp.concatenate([edge_index[0], loops])
    dst = jnp.concatenate([edge_index[1], loops])

    zero8 = jnp.zeros((H,), jnp.float32)

    def gat_layer(xin, W, att_s, att_d, bias, out_ch):
        xl = _mm(xin, W, jnp.zeros((W.shape[1],), jnp.float32))
        a_src = _mm(xl, _head_proj(att_s, out_ch), zero8)
        a_dst = _mm(xl, _head_proj(att_d, out_ch), zero8)
        agg = _gat_conv_edges(xl, a_src, a_dst, src, dst, H, out_ch, n)
        return jnp.maximum(agg + bias, 0.0)

    gg1 = _mm(x, Wn1, bn1, act="relu")
    gg2 = _mm(gg1, Wn2, bn2, act="relu")

    x1 = gat_layer(x, Wc1, as1, ad1, bc1, 16)
    x2 = gat_layer(x1, Wc2, as2, ad2, bc2, 32)
    x3 = gat_layer(x2, Wc3, as3, ad3, bc3, 32)

    xa1 = _mm(x1, Wa1, ba1, act="relu")
    xa1 = _mm(xa1, Wa2, ba2, act="relu")
    xa2 = _mm(x2, Wa3, ba3, act="relu")
    xa2 = _mm(xa2, Wa4, ba4, act="relu")

    xf = jnp.concatenate((gg2, x3, xa1, xa2), axis=1)
    xf = _mm(xf, Wf1, bf1, act="relu")
    xf = _mm(xf, Wf2, bf2, act="relu")
    xf = _mm(xf, Wf3, bf3, act="sigmoid")
    return xf


# speedup vs baseline: 21.0123x; 21.0123x over previous
import functools
import jax
import jax.numpy as jnp
from jax import lax
from jax.experimental import pallas as pl
from jax.experimental.pallas import tpu as pltpu
from jax.experimental.pallas import tpu_sc as plsc


H = 8
N_NODES = 100000
E_EDGES = 1600000
E_TOT = E_EDGES + N_NODES
BLK_A = 2048
BLK_B = 1024
BLK_C = 1024
NC_SC, NS_SC = 2, 16
NW = NC_SC * NS_SC
EP = 1703936
PER_W = EP // NW
PER_T = EP // NS_SC
ROWS_T = N_NODES // NS_SC

_mesh = plsc.VectorSubcoreMesh(core_axis_name="c", subcore_axis_name="s")
_f32 = jnp.float32
_sc_params = pltpu.CompilerParams(use_tc_tiling_on_sc=False)
import dataclasses as _dc
_sc_params_nl = _sc_params
if "needs_layout_passes" in pltpu.CompilerParams.__dataclass_fields__:
    _sc_params_nl = _dc.replace(_sc_params, needs_layout_passes=False)


def _mm_body(a_ref, w_ref, b_ref, o_ref, *, act):
    y = jnp.dot(a_ref[...], w_ref[...], preferred_element_type=_f32)
    y = y + b_ref[...]
    if act == "relu":
        y = jnp.maximum(y, 0.0)
    elif act == "sigmoid":
        y = jax.nn.sigmoid(y)
    o_ref[...] = y


def _mm(a, w, b, act="none", block=2000):
    n, k = a.shape
    m = w.shape[1]
    return pl.pallas_call(
        functools.partial(_mm_body, act=act),
        grid=(n // block,),
        in_specs=[
            pl.BlockSpec((block, k), lambda i: (i, 0)),
            pl.BlockSpec((k, m), lambda i: (0, 0)),
            pl.BlockSpec((m,), lambda i: (0,)),
        ],
        out_specs=pl.BlockSpec((block, m), lambda i: (i, 0)),
        out_shape=jax.ShapeDtypeStruct((n, m), _f32),
    )(a, w, b)


def _ex_body(asg_ref, adg_ref, o_ref):
    a = asg_ref[...]
    d = adg_ref[...]
    t = a[:, :8] + d[:, :8]
    t = jnp.where(t > 0, t, 0.2 * t)
    ex = jnp.exp(t - d[:, 8:])
    row = pl.program_id(0) * a.shape[0] + lax.broadcasted_iota(
        jnp.int32, ex.shape, 0)
    ex = jnp.where(row < E_TOT, ex, 0.0)
    o_ref[...] = jnp.concatenate([ex, jnp.zeros_like(ex)], axis=1)


def _edge_ex(asg, adg, block=8192):
    return pl.pallas_call(
        _ex_body,
        grid=(EP // block,),
        in_specs=[pl.BlockSpec((block, 16), lambda i: (i, 0))] * 2,
        out_specs=pl.BlockSpec((block, 16), lambda i: (i, 0)),
        out_shape=jax.ShapeDtypeStruct((EP, 16), _f32),
    )(asg, adg)


def _coef_body(ex_ref, d0_ref, d1_ref, o_ref):
    denom = d0_ref[...] + d1_ref[...]
    o_ref[...] = ex_ref[...] / (denom + 1e-16)


def _edge_coef(exv, dg0, dg1, block=8192):
    return pl.pallas_call(
        _coef_body,
        grid=(EP // block,),
        in_specs=[pl.BlockSpec((block, 16), lambda i: (i, 0))] * 3,
        out_specs=pl.BlockSpec((block, 16), lambda i: (i, 0)),
        out_shape=jax.ShapeDtypeStruct((EP, 16), _f32),
    )(exv, dg0, dg1)


def _bias_relu_body(x_ref, b_ref, o_ref):
    o_ref[...] = jnp.maximum(x_ref[...] + b_ref[...], 0.0)


def _bias_relu(x, b, block=2000):
    n, m = x.shape
    return pl.pallas_call(
        _bias_relu_body,
        grid=(n // block,),
        in_specs=[
            pl.BlockSpec((block, m), lambda i: (i, 0)),
            pl.BlockSpec((m,), lambda i: (0,)),
        ],
        out_specs=pl.BlockSpec((block, m), lambda i: (i, 0)),
        out_shape=jax.ShapeDtypeStruct((n, m), _f32),
    )(x, b)


def _dual_gather(tab1, tab2, idx1, idx2):
    @functools.partial(
        pl.kernel,
        mesh=_mesh,
        compiler_params=_sc_params,
        out_type=(jax.ShapeDtypeStruct((EP, 16), _f32),
                  jax.ShapeDtypeStruct((EP, 16), _f32)),
        scratch_types=[
            pltpu.VMEM((BLK_A,), jnp.int32),
            pltpu.VMEM((BLK_A,), jnp.int32),
            pltpu.VMEM((BLK_A, 16), _f32),
            pltpu.VMEM((BLK_A, 16), _f32),
            pltpu.SemaphoreType.DMA,
            pltpu.SemaphoreType.DMA,
        ],
    )
    def k(t1_hbm, t2_hbm, i1_hbm, i2_hbm, o1_hbm, o2_hbm,
          i1_v, i2_v, r1_v, r2_v, s1, s2):
        wid = lax.axis_index("s") * NC_SC + lax.axis_index("c")

        @pl.loop(0, PER_W // BLK_A)
        def _(b):
            off = wid * PER_W + b * BLK_A
            pltpu.sync_copy(i1_hbm.at[pl.ds(off, BLK_A)], i1_v)
            pltpu.sync_copy(i2_hbm.at[pl.ds(off, BLK_A)], i2_v)
            c1 = pltpu.async_copy(t1_hbm.at[i1_v], r1_v, s1)
            c2 = pltpu.async_copy(t2_hbm.at[i2_v], r2_v, s2)
            c1.wait()
            c2.wait()
            pltpu.sync_copy(r1_v, o1_hbm.at[pl.ds(off, BLK_A)])
            pltpu.sync_copy(r2_v, o2_hbm.at[pl.ds(off, BLK_A)])

    return k(tab1, tab2, idx1, idx2)


def _seg_sum16(vals, dst, zeros):
    @functools.partial(
        pl.kernel,
        mesh=_mesh,
        compiler_params=_sc_params,
        out_type=jax.ShapeDtypeStruct((NC_SC, N_NODES, 16), _f32),
        scratch_types=[
            pltpu.VMEM((BLK_B,), jnp.int32),
            pltpu.VMEM((BLK_B, 16), _f32),
            pltpu.VMEM_SHARED((N_NODES, 16), _f32),
            pltpu.SemaphoreType.DMA,
        ],
    )
    def k(v_hbm, d_hbm, z_hbm, o_hbm, d_v, v_v, acc, sem):
        core = lax.axis_index("c")
        sid = lax.axis_index("s")
        wid = sid * NC_SC + core
        pltpu.sync_copy(z_hbm, acc.at[pl.ds(sid * ROWS_T, ROWS_T)])
        plsc.subcore_barrier()

        @pl.loop(0, PER_W // BLK_B)
        def _(b):
            off = wid * PER_W + b * BLK_B
            pltpu.sync_copy(d_hbm.at[pl.ds(off, BLK_B)], d_v)
            pltpu.sync_copy(v_hbm.at[pl.ds(off, BLK_B)], v_v)
            pltpu.sync_copy(v_v, acc.at[d_v], add=True)

        plsc.subcore_barrier()
        pltpu.sync_copy(acc.at[pl.ds(sid * ROWS_T, ROWS_T)],
                        o_hbm.at[core].at[pl.ds(sid * ROWS_T, ROWS_T)])

    return k(vals, dst, zeros)


def _gat_aggregate(xlcat, coefT, src, dst, zeros, nc):
    hc = nc // H

    @functools.partial(
        pl.kernel,
        mesh=_mesh,
        compiler_params=_sc_params_nl,
        out_type=jax.ShapeDtypeStruct((nc, N_NODES, 16), _f32),
        scratch_types=[
            pltpu.VMEM((BLK_C,), jnp.int32),
            pltpu.VMEM((BLK_C,), jnp.int32),
            pltpu.VMEM((BLK_C,), _f32),
            pltpu.VMEM((BLK_C, 16), _f32),
            pltpu.VMEM_SHARED((N_NODES, 16), _f32),
            pltpu.SemaphoreType.DMA,
        ],
    )
    def k(x_hbm, c_hbm, s_hbm, d_hbm, z_hbm, o_hbm,
          s_v, d_v, c_v, r_v, acc, sem):
        core = lax.axis_index("c")
        sid = lax.axis_index("s")

        for j in range(nc // NC_SC):
            chunk = NC_SC * j + core
            head = chunk // hc
            base_row = chunk * N_NODES
            pltpu.sync_copy(z_hbm, acc.at[pl.ds(sid * ROWS_T, ROWS_T)])
            plsc.subcore_barrier()

            @pl.loop(0, PER_T // BLK_C)
            def _(b):
                off = sid * PER_T + b * BLK_C
                pltpu.sync_copy(s_hbm.at[pl.ds(off, BLK_C)], s_v)
                pltpu.sync_copy(d_hbm.at[pl.ds(off, BLK_C)], d_v)
                pltpu.sync_copy(c_hbm.at[head].at[pl.ds(off, BLK_C)], c_v)

                roff = jnp.full((16,), base_row, jnp.int32)

                @pl.loop(0, BLK_C, step=16)
                def _(i):
                    s_v.at[pl.ds(i, 16)][...] = (
                        s_v.at[pl.ds(i, 16)][...] + roff)

                pltpu.async_copy(x_hbm.at[s_v], r_v, sem).wait()

                @pl.loop(0, BLK_C, step=16)
                def _(i):
                    for jj in range(16):
                        e = i + jj
                        splat = plsc.load_gather(
                            c_v, [jnp.full((16,), e, jnp.int32)])
                        r_v.at[e][...] = r_v.at[e][...] * splat

                pltpu.sync_copy(r_v, acc.at[d_v], add=True)

            plsc.subcore_barrier()
            pltpu.sync_copy(acc.at[pl.ds(sid * ROWS_T, ROWS_T)],
                            o_hbm.at[chunk].at[pl.ds(sid * ROWS_T, ROWS_T)])
            plsc.subcore_barrier()

    return k(xlcat, coefT, src, dst, zeros)


def _head_proj(att, out_ch):
    eye = jnp.eye(H, dtype=_f32)
    return (att[:, :, None] * eye[:, None, :]).reshape(H * out_ch, H)


def _gat_layer(xin, W, att_s, att_d, bias, out_ch, src_p, dst_p, zeros):
    n = N_NODES
    d_out = H * out_ch
    nc = d_out // 16

    xl = _mm(xin, W, jnp.zeros((d_out,), _f32))
    a_src = _mm(xl, _head_proj(att_s, out_ch), jnp.zeros((H,), _f32))
    a_dst = _mm(xl, _head_proj(att_d, out_ch), jnp.zeros((H,), _f32))

    smax = jnp.max(a_src, axis=0)
    s = jnp.maximum(a_dst + smax[None, :], 0.0)
    src_tab = jnp.concatenate([a_src, jnp.zeros_like(a_src)], axis=1)
    dst_tab = jnp.concatenate([a_dst, s], axis=1)

    asg, adg = _dual_gather(src_tab, dst_tab, src_p, dst_p)
    exv = _edge_ex(asg, adg)
    dpart = _seg_sum16(exv, dst_p, zeros)
    dg0, dg1 = _dual_gather(dpart[0], dpart[1], dst_p, dst_p)
    coef = _edge_coef(exv, dg0, dg1)
    coefT = coef[:, :8].T.copy()

    xlcat = xl.reshape(n, nc, 16).transpose(1, 0, 2).reshape(nc * n, 16)
    agg = _gat_aggregate(xlcat, coefT, src_p, dst_p, zeros, nc)
    out = agg.transpose(1, 0, 2).reshape(n, d_out)
    return _bias_relu(out, bias)


def kernel(x, edge_index, Wc1, as1, ad1, bc1, Wc2, as2, ad2, bc2, Wc3, as3, ad3, bc3, Wn1, bn1, Wn2, bn2, Wa1, ba1, Wa2, ba2, Wa3, ba3, Wa4, ba4, Wf1, bf1, Wf2, bf2, Wf3, bf3):
    n = N_NODES
    loops = jnp.arange(n, dtype=edge_index.dtype)
    pad = jnp.zeros((EP - E_TOT,), edge_index.dtype)
    src_p = jnp.concatenate([edge_index[0], loops, pad])
    dst_p = jnp.concatenate([edge_index[1], loops, pad])
    zeros = jnp.zeros((ROWS_T, 16), _f32)

    gg1 = _mm(x, Wn1, bn1, act="relu")
    gg2 = _mm(gg1, Wn2, bn2, act="relu")

    x1 = _gat_layer(x, Wc1, as1, ad1, bc1, 16, src_p, dst_p, zeros)
    x2 = _gat_layer(x1, Wc2, as2, ad2, bc2, 32, src_p, dst_p, zeros)
    x3 = _gat_layer(x2, Wc3, as3, ad3, bc3, 32, src_p, dst_p, zeros)

    xa1 = _mm(x1, Wa1, ba1, act="relu")
    xa1 = _mm(xa1, Wa2, ba2, act="relu")
    xa2 = _mm(x2, Wa3, ba3, act="relu")
    xa2 = _mm(xa2, Wa4, ba4, act="relu")

    xf = jnp.concatenate((gg2, x3, xa1, xa2), axis=1)
    xf = _mm(xf, Wf1, bf1, act="relu")
    xf = _mm(xf, Wf2, bf2, act="relu")
    xf = _mm(xf, Wf3, bf3, act="sigmoid")
    return xf


# speedup vs baseline: 21.2663x; 1.0121x over previous
import functools
import jax
import jax.numpy as jnp
from jax import lax
from jax.experimental import pallas as pl
from jax.experimental.pallas import tpu as pltpu
from jax.experimental.pallas import tpu_sc as plsc


H = 8
N_NODES = 100000
E_EDGES = 1600000
E_TOT = E_EDGES + N_NODES
BLK_A = 2048
BLK_B = 1024
BLK_C = 512
NC_SC, NS_SC = 2, 16
NW = NC_SC * NS_SC
EP = 1703936
PER_W = EP // NW
PER_T = EP // NS_SC
ROWS_T = N_NODES // NS_SC

_mesh = plsc.VectorSubcoreMesh(core_axis_name="c", subcore_axis_name="s")
_f32 = jnp.float32
_sc_params = pltpu.CompilerParams(use_tc_tiling_on_sc=False)
import dataclasses as _dc
_sc_params_nl = _sc_params
if "needs_layout_passes" in pltpu.CompilerParams.__dataclass_fields__:
    _sc_params_nl = _dc.replace(_sc_params, needs_layout_passes=False)


def _mm_body(a_ref, w_ref, b_ref, o_ref, *, act):
    y = jnp.dot(a_ref[...], w_ref[...], preferred_element_type=_f32)
    y = y + b_ref[...]
    if act == "relu":
        y = jnp.maximum(y, 0.0)
    elif act == "sigmoid":
        y = jax.nn.sigmoid(y)
    o_ref[...] = y


def _mm(a, w, b, act="none", block=2000):
    n, k = a.shape
    m = w.shape[1]
    return pl.pallas_call(
        functools.partial(_mm_body, act=act),
        grid=(n // block,),
        in_specs=[
            pl.BlockSpec((block, k), lambda i: (i, 0)),
            pl.BlockSpec((k, m), lambda i: (0, 0)),
            pl.BlockSpec((m,), lambda i: (0,)),
        ],
        out_specs=pl.BlockSpec((block, m), lambda i: (i, 0)),
        out_shape=jax.ShapeDtypeStruct((n, m), _f32),
    )(a, w, b)


def _ex_body(asg_ref, adg_ref, o_ref):
    a = asg_ref[...]
    d = adg_ref[...]
    t = a[:, :8] + d[:, :8]
    t = jnp.where(t > 0, t, 0.2 * t)
    ex = jnp.exp(t - d[:, 8:])
    row = pl.program_id(0) * a.shape[0] + lax.broadcasted_iota(
        jnp.int32, ex.shape, 0)
    ex = jnp.where(row < E_TOT, ex, 0.0)
    o_ref[...] = jnp.concatenate([ex, jnp.zeros_like(ex)], axis=1)


def _edge_ex(asg, adg, block=8192):
    return pl.pallas_call(
        _ex_body,
        grid=(EP // block,),
        in_specs=[pl.BlockSpec((block, 16), lambda i: (i, 0))] * 2,
        out_specs=pl.BlockSpec((block, 16), lambda i: (i, 0)),
        out_shape=jax.ShapeDtypeStruct((EP, 16), _f32),
    )(asg, adg)


def _coef_body(ex_ref, d0_ref, d1_ref, o_ref):
    denom = d0_ref[...] + d1_ref[...]
    o_ref[...] = ex_ref[...] / (denom + 1e-16)


def _edge_coef(exv, dg0, dg1, block=8192):
    return pl.pallas_call(
        _coef_body,
        grid=(EP // block,),
        in_specs=[pl.BlockSpec((block, 16), lambda i: (i, 0))] * 3,
        out_specs=pl.BlockSpec((block, 16), lambda i: (i, 0)),
        out_shape=jax.ShapeDtypeStruct((EP, 16), _f32),
    )(exv, dg0, dg1)


def _bias_relu_body(x_ref, b_ref, o_ref):
    o_ref[...] = jnp.maximum(x_ref[...] + b_ref[...], 0.0)


def _bias_relu(x, b, block=2000):
    n, m = x.shape
    return pl.pallas_call(
        _bias_relu_body,
        grid=(n // block,),
        in_specs=[
            pl.BlockSpec((block, m), lambda i: (i, 0)),
            pl.BlockSpec((m,), lambda i: (0,)),
        ],
        out_specs=pl.BlockSpec((block, m), lambda i: (i, 0)),
        out_shape=jax.ShapeDtypeStruct((n, m), _f32),
    )(x, b)


def _dual_gather(tab1, tab2, idx1, idx2):
    @functools.partial(
        pl.kernel,
        mesh=_mesh,
        compiler_params=_sc_params,
        out_type=(jax.ShapeDtypeStruct((EP, 16), _f32),
                  jax.ShapeDtypeStruct((EP, 16), _f32)),
        scratch_types=[
            pltpu.VMEM((BLK_A,), jnp.int32),
            pltpu.VMEM((BLK_A,), jnp.int32),
            pltpu.VMEM((BLK_A, 16), _f32),
            pltpu.VMEM((BLK_A, 16), _f32),
            pltpu.SemaphoreType.DMA,
            pltpu.SemaphoreType.DMA,
        ],
    )
    def k(t1_hbm, t2_hbm, i1_hbm, i2_hbm, o1_hbm, o2_hbm,
          i1_v, i2_v, r1_v, r2_v, s1, s2):
        wid = lax.axis_index("s") * NC_SC + lax.axis_index("c")

        @pl.loop(0, PER_W // BLK_A)
        def _(b):
            off = wid * PER_W + b * BLK_A
            pltpu.sync_copy(i1_hbm.at[pl.ds(off, BLK_A)], i1_v)
            pltpu.sync_copy(i2_hbm.at[pl.ds(off, BLK_A)], i2_v)
            c1 = pltpu.async_copy(t1_hbm.at[i1_v], r1_v, s1)
            c2 = pltpu.async_copy(t2_hbm.at[i2_v], r2_v, s2)
            c1.wait()
            c2.wait()
            pltpu.sync_copy(r1_v, o1_hbm.at[pl.ds(off, BLK_A)])
            pltpu.sync_copy(r2_v, o2_hbm.at[pl.ds(off, BLK_A)])

    return k(tab1, tab2, idx1, idx2)


def _seg_sum16(vals, dst, zeros):
    @functools.partial(
        pl.kernel,
        mesh=_mesh,
        compiler_params=_sc_params,
        out_type=jax.ShapeDtypeStruct((NC_SC, N_NODES, 16), _f32),
        scratch_types=[
            pltpu.VMEM((BLK_B,), jnp.int32),
            pltpu.VMEM((BLK_B, 16), _f32),
            pltpu.VMEM_SHARED((N_NODES, 16), _f32),
            pltpu.SemaphoreType.DMA,
        ],
    )
    def k(v_hbm, d_hbm, z_hbm, o_hbm, d_v, v_v, acc, sem):
        core = lax.axis_index("c")
        sid = lax.axis_index("s")
        wid = sid * NC_SC + core
        pltpu.sync_copy(z_hbm, acc.at[pl.ds(sid * ROWS_T, ROWS_T)])
        plsc.subcore_barrier()

        @pl.loop(0, PER_W // BLK_B)
        def _(b):
            off = wid * PER_W + b * BLK_B
            pltpu.sync_copy(d_hbm.at[pl.ds(off, BLK_B)], d_v)
            pltpu.sync_copy(v_hbm.at[pl.ds(off, BLK_B)], v_v)
            pltpu.sync_copy(v_v, acc.at[d_v], add=True)

        plsc.subcore_barrier()
        pltpu.sync_copy(acc.at[pl.ds(sid * ROWS_T, ROWS_T)],
                        o_hbm.at[core].at[pl.ds(sid * ROWS_T, ROWS_T)])

    return k(vals, dst, zeros)


def _gat_aggregate(xlcat, coefT, src, dst, zeros, nc):
    hc = nc // H
    nb = PER_T // BLK_C

    @functools.partial(
        pl.kernel,
        mesh=_mesh,
        compiler_params=_sc_params_nl,
        out_type=jax.ShapeDtypeStruct((nc, N_NODES, 16), _f32),
        scratch_types=[
            pltpu.VMEM((BLK_C,), jnp.int32),
            pltpu.VMEM((BLK_C,), jnp.int32),
            pltpu.VMEM((BLK_C,), jnp.int32),
            pltpu.VMEM((BLK_C,), jnp.int32),
            pltpu.VMEM((BLK_C,), _f32),
            pltpu.VMEM((BLK_C,), _f32),
            pltpu.VMEM((BLK_C, 16), _f32),
            pltpu.VMEM((BLK_C, 16), _f32),
            pltpu.VMEM_SHARED((N_NODES, 16), _f32),
            pltpu.SemaphoreType.DMA,
            pltpu.SemaphoreType.DMA,
        ],
    )
    def k(x_hbm, c_hbm, s_hbm, d_hbm, z_hbm, o_hbm,
          s_v0, s_v1, d_v0, d_v1, c_v0, c_v1, r_v0, r_v1, acc, sem0, sem1):
        core = lax.axis_index("c")
        sid = lax.axis_index("s")
        s_v = (s_v0, s_v1)
        d_v = (d_v0, d_v1)
        c_v = (c_v0, c_v1)
        r_v = (r_v0, r_v1)
        sem = (sem0, sem1)

        for j in range(nc // NC_SC):
            chunk = NC_SC * j + core
            head = chunk // hc
            base_row = chunk * N_NODES
            roff = jnp.full((16,), base_row, jnp.int32)

            def stage(b, p):
                off = sid * PER_T + b * BLK_C
                pltpu.sync_copy(s_hbm.at[pl.ds(off, BLK_C)], s_v[p])
                pltpu.sync_copy(d_hbm.at[pl.ds(off, BLK_C)], d_v[p])
                pltpu.sync_copy(c_hbm.at[head].at[pl.ds(off, BLK_C)], c_v[p])

                @pl.loop(0, BLK_C, step=16)
                def _(i):
                    s_v[p].at[pl.ds(i, 16)][...] = (
                        s_v[p].at[pl.ds(i, 16)][...] + roff)

                pltpu.async_copy(x_hbm.at[s_v[p]], r_v[p], sem[p])

            def consume(p):
                pltpu.make_async_copy(x_hbm.at[s_v[p]], r_v[p], sem[p]).wait()

                @pl.loop(0, BLK_C, step=16)
                def _(i):
                    for jj in range(16):
                        e = i + jj
                        splat = plsc.load_gather(
                            c_v[p], [jnp.full((16,), e, jnp.int32)])
                        r_v[p].at[e][...] = r_v[p].at[e][...] * splat

                pltpu.sync_copy(r_v[p], acc.at[d_v[p]], add=True)

            pltpu.sync_copy(z_hbm, acc.at[pl.ds(sid * ROWS_T, ROWS_T)])
            plsc.subcore_barrier()

            stage(0, 0)

            @pl.loop(0, nb // 2)
            def _(kk):
                b = 2 * kk
                stage_b1 = b + 1
                stage(stage_b1, 1)
                consume(0)
                nxt = lax.rem(b + 2, nb)
                stage(nxt, 0)
                consume(1)

            pltpu.make_async_copy(x_hbm.at[s_v[0]], r_v[0], sem[0]).wait()

            plsc.subcore_barrier()
            pltpu.sync_copy(acc.at[pl.ds(sid * ROWS_T, ROWS_T)],
                            o_hbm.at[chunk].at[pl.ds(sid * ROWS_T, ROWS_T)])
            plsc.subcore_barrier()

    return k(xlcat, coefT, src, dst, zeros)


def _head_proj(att, out_ch):
    eye = jnp.eye(H, dtype=_f32)
    return (att[:, :, None] * eye[:, None, :]).reshape(H * out_ch, H)


def _gat_layer(xin, W, att_s, att_d, bias, out_ch, src_p, dst_p, zeros):
    n = N_NODES
    d_out = H * out_ch
    nc = d_out // 16

    xl = _mm(xin, W, jnp.zeros((d_out,), _f32))
    a_src = _mm(xl, _head_proj(att_s, out_ch), jnp.zeros((H,), _f32))
    a_dst = _mm(xl, _head_proj(att_d, out_ch), jnp.zeros((H,), _f32))

    smax = jnp.max(a_src, axis=0)
    s = jnp.maximum(a_dst + smax[None, :], 0.0)
    src_tab = jnp.concatenate([a_src, jnp.zeros_like(a_src)], axis=1)
    dst_tab = jnp.concatenate([a_dst, s], axis=1)

    asg, adg = _dual_gather(src_tab, dst_tab, src_p, dst_p)
    exv = _edge_ex(asg, adg)
    dpart = _seg_sum16(exv, dst_p, zeros)
    dg0, dg1 = _dual_gather(dpart[0], dpart[1], dst_p, dst_p)
    coef = _edge_coef(exv, dg0, dg1)
    coefT = coef[:, :8].T.copy()

    xlcat = xl.reshape(n, nc, 16).transpose(1, 0, 2).reshape(nc * n, 16)
    agg = _gat_aggregate(xlcat, coefT, src_p, dst_p, zeros, nc)
    out = agg.transpose(1, 0, 2).reshape(n, d_out)
    return _bias_relu(out, bias)


def kernel(x, edge_index, Wc1, as1, ad1, bc1, Wc2, as2, ad2, bc2, Wc3, as3, ad3, bc3, Wn1, bn1, Wn2, bn2, Wa1, ba1, Wa2, ba2, Wa3, ba3, Wa4, ba4, Wf1, bf1, Wf2, bf2, Wf3, bf3):
    n = N_NODES
    loops = jnp.arange(n, dtype=edge_index.dtype)
    pad = jnp.zeros((EP - E_TOT,), edge_index.dtype)
    src_p = jnp.concatenate([edge_index[0], loops, pad])
    dst_p = jnp.concatenate([edge_index[1], loops, pad])
    zeros = jnp.zeros((ROWS_T, 16), _f32)

    gg1 = _mm(x, Wn1, bn1, act="relu")
    gg2 = _mm(gg1, Wn2, bn2, act="relu")

    x1 = _gat_layer(x, Wc1, as1, ad1, bc1, 16, src_p, dst_p, zeros)
    x2 = _gat_layer(x1, Wc2, as2, ad2, bc2, 32, src_p, dst_p, zeros)
    x3 = _gat_layer(x2, Wc3, as3, ad3, bc3, 32, src_p, dst_p, zeros)

    xa1 = _mm(x1, Wa1, ba1, act="relu")
    xa1 = _mm(xa1, Wa2, ba2, act="relu")
    xa2 = _mm(x2, Wa3, ba3, act="relu")
    xa2 = _mm(xa2, Wa4, ba4, act="relu")

    xf = jnp.concatenate((gg2, x3, xa1, xa2), axis=1)
    xf = _mm(xf, Wf1, bf1, act="relu")
    xf = _mm(xf, Wf2, bf2, act="relu")
    xf = _mm(xf, Wf3, bf3, act="sigmoid")
    return xf
